# Initial kernel scaffold; baseline (speedup 1.0000x reference)
import jax
import jax.numpy as jnp
from jax.experimental import pallas as pl


def kernel(logits, in_lens):
    raise NotImplementedError("write your pallas kernel here")


# speedup vs baseline: 4.3936x; 4.3936x over previous
import functools

import jax
import jax.numpy as jnp
from jax.experimental import pallas as pl

T, N, V = 2048, 16, 1024
BLANK = 0
TB = 128


def _dense_body(lens_ref, x_ref, amax_ref, msum_ref):
    i = pl.program_id(0)
    x = x_ref[...]
    m = jnp.max(x, axis=-1)
    v = jax.lax.broadcasted_iota(jnp.int32, x.shape, 2)
    a = jnp.min(jnp.where(x == m[..., None], v, V), axis=-1)
    s = jnp.sum(jnp.exp(x - m[..., None]), axis=-1)
    mlp = -jnp.log(s)
    amax_ref[...] = a
    t = i * TB + jax.lax.broadcasted_iota(jnp.int32, (TB, 1), 0)
    mask = t < lens_ref[...]
    contrib = jnp.sum(jnp.where(mask, mlp, 0.0), axis=0, keepdims=True)

    @pl.when(i == 0)
    def _init():
        msum_ref[...] = jnp.zeros_like(msum_ref)

    msum_ref[...] += contrib


def _shift_down(x, k, fill):
    pad = jnp.full((k,) + x.shape[1:], fill, dtype=x.dtype)
    return jnp.concatenate([pad, x[:-k]], axis=0)


def _shift_up(x, k, fill):
    pad = jnp.full((k,) + x.shape[1:], fill, dtype=x.dtype)
    return jnp.concatenate([x[k:], pad], axis=0)


def _compact_body(a_ref, lens_ref, paths_ref, olens_ref):
    a = a_ref[...]
    lens = lens_ref[...]
    t = jax.lax.broadcasted_iota(jnp.int32, (T, 1), 0)
    prev = _shift_down(a, 1, 0)
    keep = (a != BLANK) & ((a != prev) | (t == 0)) & (t < lens)
    c = keep.astype(jnp.int32)
    k = 1
    while k < T:
        c = c + _shift_down(c, k, 0)
        k *= 2
    out_lens = c[-1:, :]
    olens_ref[...] = out_lens
    val = a
    occ = keep.astype(jnp.int32)
    rem = t + 1 - c
    k = 1
    while k < T:
        want = occ * ((rem & k) != 0).astype(jnp.int32)
        want_s = _shift_up(want, k, 0) != 0
        val_s = _shift_up(val, k, 0)
        rem_s = _shift_up(rem, k, 0)
        val = jnp.where(want_s, val_s, val)
        rem = jnp.where(want_s, rem_s - k, rem)
        occ = jnp.where(want_s, 1, occ * (1 - want))
        k *= 2
    paths_ref[...] = jnp.where(t < out_lens, val, a)


@jax.jit
def kernel(logits, in_lens):
    lens2d = in_lens.astype(jnp.int32).reshape(1, N)
    amax, msum = pl.pallas_call(
        _dense_body,
        grid=(T // TB,),
        in_specs=[
            pl.BlockSpec((1, N), lambda i: (0, 0)),
            pl.BlockSpec((TB, N, V), lambda i: (i, 0, 0)),
        ],
        out_specs=[
            pl.BlockSpec((TB, N), lambda i: (i, 0)),
            pl.BlockSpec((1, N), lambda i: (0, 0)),
        ],
        out_shape=[
            jax.ShapeDtypeStruct((T, N), jnp.int32),
            jax.ShapeDtypeStruct((1, N), jnp.float32),
        ],
    )(lens2d, logits)
    paths, out_lens = pl.pallas_call(
        _compact_body,
        in_specs=[
            pl.BlockSpec((T, N), lambda: (0, 0)),
            pl.BlockSpec((1, N), lambda: (0, 0)),
        ],
        out_specs=[
            pl.BlockSpec((T, N), lambda: (0, 0)),
            pl.BlockSpec((1, N), lambda: (0, 0)),
        ],
        out_shape=[
            jax.ShapeDtypeStruct((T, N), jnp.int32),
            jax.ShapeDtypeStruct((1, N), jnp.int32),
        ],
    )(amax, lens2d)
    return msum.reshape(N), paths, out_lens.reshape(N)
